# baseline (speedup 1.0000x reference)
import functools

import jax
import jax.numpy as jnp
from jax import lax
from jax.experimental import pallas as pl
from jax.experimental.pallas import tpu as pltpu
from jax.experimental.pallas import tpu_sc as plsc

N_BLK = 1000
E_BLK = 640

_NC, _NS = 2, 16
_NW = _NC * _NS
_GK = 128
_GCH = 40
E_PAD = _NW * _GCH * _GK


def _gather_body(a_hbm, q_hbm, src_hbm, dst_hbm, asrc_hbm, qd_hbm,
                 idx_s, idx_d, buf_a, buf_q, sem_a, sem_q):
    wid = lax.axis_index("s") * _NC + lax.axis_index("c")

    def chunk(j, carry):
        e0 = (wid * _GCH + j) * _GK
        pltpu.sync_copy(src_hbm.at[pl.ds(e0, _GK)], idx_s)
        pltpu.sync_copy(dst_hbm.at[pl.ds(e0, _GK)], idx_d)
        ca = pltpu.async_copy(a_hbm.at[idx_s], buf_a, sem_a)
        cq = pltpu.async_copy(q_hbm.at[idx_d], buf_q, sem_q)
        ca.wait()
        cq.wait()
        pltpu.sync_copy(buf_a, asrc_hbm.at[pl.ds(e0, _GK)])
        pltpu.sync_copy(buf_q, qd_hbm.at[pl.ds(e0, _GK)])
        return carry

    lax.fori_loop(0, _GCH, chunk, 0)


def _edge_gather(A, Q, srcp, dstp):
    c = A.shape[1]
    fn = pl.kernel(
        _gather_body,
        mesh=plsc.VectorSubcoreMesh(core_axis_name="c", subcore_axis_name="s"),
        out_type=[
            jax.ShapeDtypeStruct((E_PAD, c), jnp.float32),
            jax.ShapeDtypeStruct((E_PAD, c), jnp.float32),
        ],
        scratch_types=[
            pltpu.VMEM((_GK,), jnp.int32),
            pltpu.VMEM((_GK,), jnp.int32),
            pltpu.VMEM((_GK, c), jnp.float32),
            pltpu.VMEM((_GK, c), jnp.float32),
            pltpu.SemaphoreType.DMA,
            pltpu.SemaphoreType.DMA,
        ],
    )
    return fn(A, Q, srcp, dstp)


def _node_prep_kernel(ivf_ref, xyzp_ref, w1a_ref, w1b_ref, b1_ref, a_ref, q_ref):
    q = jnp.dot(xyzp_ref[...], w1b_ref[...], preferred_element_type=jnp.float32)
    a = jnp.dot(ivf_ref[...], w1a_ref[...], preferred_element_type=jnp.float32)
    a_ref[...] = a + q + b1_ref[...]
    q_ref[...] = q


def _node_prep(ivf, xyzp, w1aT, w1bpT, b1):
    n = ivf.shape[0]
    c = ivf.shape[1]
    grid = (n // N_BLK,)
    return pl.pallas_call(
        _node_prep_kernel,
        grid=grid,
        in_specs=[
            pl.BlockSpec((N_BLK, c), lambda i: (i, 0)),
            pl.BlockSpec((N_BLK, 128), lambda i: (i, 0)),
            pl.BlockSpec((c, c), lambda i: (0, 0)),
            pl.BlockSpec((128, c), lambda i: (0, 0)),
            pl.BlockSpec((1, c), lambda i: (0, 0)),
        ],
        out_specs=[
            pl.BlockSpec((N_BLK, c), lambda i: (i, 0)),
            pl.BlockSpec((N_BLK, c), lambda i: (i, 0)),
        ],
        out_shape=[
            jax.ShapeDtypeStruct((n, c), jnp.float32),
            jax.ShapeDtypeStruct((n, c), jnp.float32),
        ],
    )(ivf, xyzp, w1aT, w1bpT, b1)


_NEG = -3.0e38
_NR = 320
_SCH = 1250


def _scatter_body(h1_hbm, dst_hbm, out_hbm, dstbuf, rowsbuf, agg_l, ridx, sem):
    wid = lax.axis_index("s") * _NC + lax.axis_index("c")
    lo = wid * _NR

    def init(i, carry):
        agg_l[pl.ds(i * 16, 16)] = jnp.full((16,), _NEG, dtype=jnp.float32)
        return carry

    lax.fori_loop(0, _NR * 256 // 16, init, 0)

    def chunk(j, carry):
        e0 = j * _GK
        pltpu.sync_copy(dst_hbm.at[pl.ds(e0, _GK)], dstbuf)

        def grp(t, cnt_t):
            dvec = dstbuf[pl.ds(t * 16, 16)]
            cnt2 = cnt_t
            for l in range(16):
                d = dvec[l]
                hit = (d >= lo) & (d < lo + _NR)

                @pl.when(hit)
                def _(d=d, l=l, t=t, cnt2=cnt2):
                    pltpu.async_copy(
                        h1_hbm.at[pl.ds((e0 + t * 16 + l) * 256, 256)],
                        rowsbuf.at[pl.ds(cnt2 * 256, 256)], sem)
                    ridx[cnt2] = d - lo

                cnt2 = jnp.where(hit, cnt2 + 1, cnt2)
            return cnt2

        cnt = lax.fori_loop(0, _GK // 16, grp, 0)

        def drain(k, carry2):
            pltpu.make_async_copy(h1_hbm.at[pl.ds(0, 256)],
                                  rowsbuf.at[pl.ds(0, 256)], sem).wait()
            return carry2

        lax.fori_loop(0, cnt, drain, 0)

        def acc(k, carry3):
            r = ridx[k]
            for i in range(16):
                cur = agg_l[pl.ds(r * 256 + i * 16, 16)]
                v = rowsbuf[pl.ds(k * 256 + i * 16, 16)]
                agg_l[pl.ds(r * 256 + i * 16, 16)] = jnp.where(v > cur, v, cur)
            return carry3

        lax.fori_loop(0, cnt, acc, 0)
        return carry

    lax.fori_loop(0, _SCH, chunk, 0)
    pltpu.sync_copy(agg_l, out_hbm.at[pl.ds(wid * _NR * 256, _NR * 256)])


def _scatter_max(h, dst):
    h1 = h.reshape(E_PAD * 256)
    fn = pl.kernel(
        _scatter_body,
        mesh=plsc.VectorSubcoreMesh(core_axis_name="c", subcore_axis_name="s"),
        out_type=jax.ShapeDtypeStruct((32 * _NR * 256,), jnp.float32),
        scratch_types=[
            pltpu.VMEM((_GK,), jnp.int32),
            pltpu.VMEM((_GK * 256,), jnp.float32),
            pltpu.VMEM((_NR * 256,), jnp.float32),
            pltpu.SMEM((_GK,), jnp.int32),
            pltpu.SemaphoreType.DMA,
        ],
    )
    raw = fn(h1, dst)
    return raw.reshape(32 * _NR, 256)[:10000]


def _edge_ln_kernel(asrc_ref, qd_ref, g_ref, be_ref, h_ref):
    x = jnp.maximum(asrc_ref[...] - qd_ref[...], 0.0)
    mu = jnp.mean(x, axis=-1, keepdims=True)
    xc = x - mu
    var = jnp.mean(xc * xc, axis=-1, keepdims=True)
    h_ref[...] = xc * jax.lax.rsqrt(var + 1e-5) * g_ref[...] + be_ref[...]


def _edge_ln(asrc, qd, g1, be1):
    e, c = asrc.shape
    return pl.pallas_call(
        _edge_ln_kernel,
        grid=(e // E_BLK,),
        in_specs=[
            pl.BlockSpec((E_BLK, c), lambda i: (i, 0)),
            pl.BlockSpec((E_BLK, c), lambda i: (i, 0)),
            pl.BlockSpec((1, c), lambda i: (0, 0)),
            pl.BlockSpec((1, c), lambda i: (0, 0)),
        ],
        out_specs=pl.BlockSpec((E_BLK, c), lambda i: (i, 0)),
        out_shape=jax.ShapeDtypeStruct((e, c), jnp.float32),
    )(asrc, qd, g1, be1)


def _update_kernel(agg_ref, ivf_ref, w2_ref, b2_ref, g_ref, be_ref, out_ref):
    a = agg_ref[...]
    a = jnp.where(a < -1.0e38, 0.0, a)
    u = jnp.dot(a, w2_ref[...], preferred_element_type=jnp.float32)
    u = jnp.maximum(u + b2_ref[...], 0.0)
    mu = jnp.mean(u, axis=-1, keepdims=True)
    uc = u - mu
    var = jnp.mean(uc * uc, axis=-1, keepdims=True)
    out_ref[...] = uc * jax.lax.rsqrt(var + 1e-5) * g_ref[...] + be_ref[...] + ivf_ref[...]


def _update(agg, ivf, w2T, b2, g2, be2):
    n, c = agg.shape
    return pl.pallas_call(
        _update_kernel,
        grid=(n // N_BLK,),
        in_specs=[
            pl.BlockSpec((N_BLK, c), lambda i: (i, 0)),
            pl.BlockSpec((N_BLK, c), lambda i: (i, 0)),
            pl.BlockSpec((c, c), lambda i: (0, 0)),
            pl.BlockSpec((1, c), lambda i: (0, 0)),
            pl.BlockSpec((1, c), lambda i: (0, 0)),
            pl.BlockSpec((1, c), lambda i: (0, 0)),
        ],
        out_specs=pl.BlockSpec((N_BLK, c), lambda i: (i, 0)),
        out_shape=jax.ShapeDtypeStruct((n, c), jnp.float32),
    )(agg, ivf, w2T, b2, g2, be2)


def kernel(xyz, features, edges, W1, b1, g1, be1, W2, b2, g2, be2):
    n = xyz.shape[1]
    c = features.shape[1]
    ivf = features[0].T
    src = edges[0, :, 0]
    dst = edges[0, :, 1]
    xyzp = jnp.pad(xyz[0], ((0, 0), (0, 125)))
    w1aT = W1[:, :c].T
    w1bpT = jnp.pad(W1[:, c:].T, ((0, 125), (0, 0)))

    A, Q = _node_prep(ivf, xyzp, w1aT, w1bpT, b1[None, :])

    e = src.shape[0]
    srcp = jnp.pad(src, (0, E_PAD - e))
    dstp = jnp.pad(dst, (0, E_PAD - e))
    asrc, qd = _edge_gather(A, Q, srcp, dstp)
    h = _edge_ln(asrc, qd, g1[None, :], be1[None, :])
    agg = _scatter_max(h, dst)

    u = _update(agg, ivf, W2.T, b2[None, :], g2[None, :], be2[None, :])
    return u.T[None]


# speedup vs baseline: 1.1431x; 1.1431x over previous
import functools

import jax
import jax.numpy as jnp
from jax import lax
from jax.experimental import pallas as pl
from jax.experimental.pallas import tpu as pltpu
from jax.experimental.pallas import tpu_sc as plsc

N_BLK = 1000
E_BLK = 640

_NC, _NS = 2, 16
_NW = _NC * _NS
_GK = 128
_GCH = 40
E_PAD = _NW * _GCH * _GK


def _gather_body(a_hbm, q_hbm, src_hbm, dst_hbm, asrc_hbm, qd_hbm,
                 idx_s, idx_d, buf_a, buf_q, sem_a, sem_q):
    wid = lax.axis_index("s") * _NC + lax.axis_index("c")

    def chunk(j, carry):
        e0 = (wid * _GCH + j) * _GK
        pltpu.sync_copy(src_hbm.at[pl.ds(e0, _GK)], idx_s)
        pltpu.sync_copy(dst_hbm.at[pl.ds(e0, _GK)], idx_d)
        ca = pltpu.async_copy(a_hbm.at[idx_s], buf_a, sem_a)
        cq = pltpu.async_copy(q_hbm.at[idx_d], buf_q, sem_q)
        ca.wait()
        cq.wait()
        pltpu.sync_copy(buf_a, asrc_hbm.at[pl.ds(e0, _GK)])
        pltpu.sync_copy(buf_q, qd_hbm.at[pl.ds(e0, _GK)])
        return carry

    lax.fori_loop(0, _GCH, chunk, 0)


def _edge_gather(A, Q, srcp, dstp):
    c = A.shape[1]
    fn = pl.kernel(
        _gather_body,
        mesh=plsc.VectorSubcoreMesh(core_axis_name="c", subcore_axis_name="s"),
        out_type=[
            jax.ShapeDtypeStruct((E_PAD, c), jnp.float32),
            jax.ShapeDtypeStruct((E_PAD, c), jnp.float32),
        ],
        scratch_types=[
            pltpu.VMEM((_GK,), jnp.int32),
            pltpu.VMEM((_GK,), jnp.int32),
            pltpu.VMEM((_GK, c), jnp.float32),
            pltpu.VMEM((_GK, c), jnp.float32),
            pltpu.SemaphoreType.DMA,
            pltpu.SemaphoreType.DMA,
        ],
    )
    return fn(A, Q, srcp, dstp)


def _node_prep_kernel(ivf_ref, xyzp_ref, w1a_ref, w1b_ref, b1_ref, a_ref, q_ref):
    q = jnp.dot(xyzp_ref[...], w1b_ref[...], preferred_element_type=jnp.float32)
    a = jnp.dot(ivf_ref[...], w1a_ref[...], preferred_element_type=jnp.float32)
    a_ref[...] = a + q + b1_ref[...]
    q_ref[...] = q


def _node_prep(ivf, xyzp, w1aT, w1bpT, b1):
    n = ivf.shape[0]
    c = ivf.shape[1]
    grid = (n // N_BLK,)
    return pl.pallas_call(
        _node_prep_kernel,
        grid=grid,
        in_specs=[
            pl.BlockSpec((N_BLK, c), lambda i: (i, 0)),
            pl.BlockSpec((N_BLK, 128), lambda i: (i, 0)),
            pl.BlockSpec((c, c), lambda i: (0, 0)),
            pl.BlockSpec((128, c), lambda i: (0, 0)),
            pl.BlockSpec((1, c), lambda i: (0, 0)),
        ],
        out_specs=[
            pl.BlockSpec((N_BLK, c), lambda i: (i, 0)),
            pl.BlockSpec((N_BLK, c), lambda i: (i, 0)),
        ],
        out_shape=[
            jax.ShapeDtypeStruct((n, c), jnp.float32),
            jax.ShapeDtypeStruct((n, c), jnp.float32),
        ],
    )(ivf, xyzp, w1aT, w1bpT, b1)


_NEG = -3.0e38
_NR = 320
_DCH = 640
_SCH = 250


def _scatter_body(h1_hbm, dst_hbm, out_hbm, dstbuf, rowsbuf, agg_l, ridx, sem):
    wid = lax.axis_index("s") * _NC + lax.axis_index("c")
    lo = wid * _NR

    def init(i, carry):
        agg_l[pl.ds(i * 16, 16)] = jnp.full((16,), _NEG, dtype=jnp.float32)
        return carry

    lax.fori_loop(0, _NR * 256 // 16, init, 0)

    def chunk(j, carry):
        e0 = j * _DCH
        pltpu.sync_copy(dst_hbm.at[pl.ds(e0, _DCH)], dstbuf)

        def sub(s, carry_s):
            def grp(t, cnt_t):
                dvec = dstbuf[pl.ds(s * _GK + t * 16, 16)]
                c2 = cnt_t
                for l in range(16):
                    d = dvec[l]
                    hit = (d >= lo) & (d < lo + _NR)

                    @pl.when(hit)
                    def _(d=d, l=l, c2=c2):
                        pltpu.async_copy(
                            h1_hbm.at[pl.ds(
                                (e0 + s * _GK + t * 16 + l) * 256, 256)],
                            rowsbuf.at[pl.ds(c2 * 256, 256)], sem)
                        ridx[c2] = d - lo

                    c2 = jnp.where(hit, c2 + 1, c2)
                return c2

            cnt = lax.fori_loop(0, _GK // 16, grp, 0)

            def drain(k, carry2):
                pltpu.make_async_copy(h1_hbm.at[pl.ds(0, 256)],
                                      rowsbuf.at[pl.ds(0, 256)], sem).wait()
                return carry2

            lax.fori_loop(0, cnt, drain, 0)

            def acc(k, carry3):
                r = ridx[k]
                for i in range(16):
                    cur = agg_l[pl.ds(r * 256 + i * 16, 16)]
                    v = rowsbuf[pl.ds(k * 256 + i * 16, 16)]
                    agg_l[pl.ds(r * 256 + i * 16, 16)] = jnp.where(
                        v > cur, v, cur)
                return carry3

            lax.fori_loop(0, cnt, acc, 0)
            return carry_s

        lax.fori_loop(0, _DCH // _GK, sub, 0)
        return carry

    lax.fori_loop(0, _SCH, chunk, 0)
    pltpu.sync_copy(agg_l, out_hbm.at[pl.ds(wid * _NR * 256, _NR * 256)])


def _scatter_max(h, dst):
    h1 = h.reshape(E_PAD * 256)
    fn = pl.kernel(
        _scatter_body,
        mesh=plsc.VectorSubcoreMesh(core_axis_name="c", subcore_axis_name="s"),
        out_type=jax.ShapeDtypeStruct((32 * _NR * 256,), jnp.float32),
        scratch_types=[
            pltpu.VMEM((_DCH,), jnp.int32),
            pltpu.VMEM((_GK * 256,), jnp.float32),
            pltpu.VMEM((_NR * 256,), jnp.float32),
            pltpu.SMEM((_GK,), jnp.int32),
            pltpu.SemaphoreType.DMA,
        ],
    )
    raw = fn(h1, dst)
    return raw.reshape(32 * _NR, 256)[:10000]


def _edge_ln_kernel(asrc_ref, qd_ref, g_ref, be_ref, h_ref):
    x = jnp.maximum(asrc_ref[...] - qd_ref[...], 0.0)
    mu = jnp.mean(x, axis=-1, keepdims=True)
    xc = x - mu
    var = jnp.mean(xc * xc, axis=-1, keepdims=True)
    h_ref[...] = xc * jax.lax.rsqrt(var + 1e-5) * g_ref[...] + be_ref[...]


def _edge_ln(asrc, qd, g1, be1):
    e, c = asrc.shape
    return pl.pallas_call(
        _edge_ln_kernel,
        grid=(e // E_BLK,),
        in_specs=[
            pl.BlockSpec((E_BLK, c), lambda i: (i, 0)),
            pl.BlockSpec((E_BLK, c), lambda i: (i, 0)),
            pl.BlockSpec((1, c), lambda i: (0, 0)),
            pl.BlockSpec((1, c), lambda i: (0, 0)),
        ],
        out_specs=pl.BlockSpec((E_BLK, c), lambda i: (i, 0)),
        out_shape=jax.ShapeDtypeStruct((e, c), jnp.float32),
    )(asrc, qd, g1, be1)


def _update_kernel(agg_ref, ivf_ref, w2_ref, b2_ref, g_ref, be_ref, out_ref):
    a = agg_ref[...]
    a = jnp.where(a < -1.0e38, 0.0, a)
    u = jnp.dot(a, w2_ref[...], preferred_element_type=jnp.float32)
    u = jnp.maximum(u + b2_ref[...], 0.0)
    mu = jnp.mean(u, axis=-1, keepdims=True)
    uc = u - mu
    var = jnp.mean(uc * uc, axis=-1, keepdims=True)
    out_ref[...] = uc * jax.lax.rsqrt(var + 1e-5) * g_ref[...] + be_ref[...] + ivf_ref[...]


def _update(agg, ivf, w2T, b2, g2, be2):
    n, c = agg.shape
    return pl.pallas_call(
        _update_kernel,
        grid=(n // N_BLK,),
        in_specs=[
            pl.BlockSpec((N_BLK, c), lambda i: (i, 0)),
            pl.BlockSpec((N_BLK, c), lambda i: (i, 0)),
            pl.BlockSpec((c, c), lambda i: (0, 0)),
            pl.BlockSpec((1, c), lambda i: (0, 0)),
            pl.BlockSpec((1, c), lambda i: (0, 0)),
            pl.BlockSpec((1, c), lambda i: (0, 0)),
        ],
        out_specs=pl.BlockSpec((N_BLK, c), lambda i: (i, 0)),
        out_shape=jax.ShapeDtypeStruct((n, c), jnp.float32),
    )(agg, ivf, w2T, b2, g2, be2)


def kernel(xyz, features, edges, W1, b1, g1, be1, W2, b2, g2, be2):
    n = xyz.shape[1]
    c = features.shape[1]
    ivf = features[0].T
    src = edges[0, :, 0]
    dst = edges[0, :, 1]
    xyzp = jnp.pad(xyz[0], ((0, 0), (0, 125)))
    w1aT = W1[:, :c].T
    w1bpT = jnp.pad(W1[:, c:].T, ((0, 125), (0, 0)))

    A, Q = _node_prep(ivf, xyzp, w1aT, w1bpT, b1[None, :])

    e = src.shape[0]
    srcp = jnp.pad(src, (0, E_PAD - e))
    dstp = jnp.pad(dst, (0, E_PAD - e))
    asrc, qd = _edge_gather(A, Q, srcp, dstp)
    h = _edge_ln(asrc, qd, g1[None, :], be1[None, :])
    agg = _scatter_max(h, dst)

    u = _update(agg, ivf, W2.T, b2[None, :], g2[None, :], be2[None, :])
    return u.T[None]


# speedup vs baseline: 1.1542x; 1.0097x over previous
import functools

import jax
import jax.numpy as jnp
from jax import lax
from jax.experimental import pallas as pl
from jax.experimental.pallas import tpu as pltpu
from jax.experimental.pallas import tpu_sc as plsc

N_BLK = 1000
E_BLK = 640

_NC, _NS = 2, 16
_NW = _NC * _NS
_GK = 128
_GCH = 40
E_PAD = _NW * _GCH * _GK


def _gather_body(a_hbm, q_hbm, src_hbm, dst_hbm, diff_hbm,
                 idx_s, idx_d, buf_a, buf_q, sem_a, sem_q):
    wid = lax.axis_index("s") * _NC + lax.axis_index("c")

    def chunk(j, carry):
        e0 = (wid * _GCH + j) * _GK
        pltpu.sync_copy(src_hbm.at[pl.ds(e0, _GK)], idx_s)
        pltpu.sync_copy(dst_hbm.at[pl.ds(e0, _GK)], idx_d)
        ca = pltpu.async_copy(a_hbm.at[idx_s], buf_a, sem_a)
        cq = pltpu.async_copy(q_hbm.at[idx_d], buf_q, sem_q)
        ca.wait()
        cq.wait()

        def sub(r, carry2):
            for i in range(16):
                a = buf_a[r, pl.ds(i * 16, 16)]
                q = buf_q[r, pl.ds(i * 16, 16)]
                buf_a[r, pl.ds(i * 16, 16)] = a - q
            return carry2

        lax.fori_loop(0, _GK, sub, 0)
        pltpu.sync_copy(buf_a, diff_hbm.at[pl.ds(e0, _GK)])
        return carry

    lax.fori_loop(0, _GCH, chunk, 0)


def _edge_gather(A, Q, srcp, dstp):
    c = A.shape[1]
    fn = pl.kernel(
        _gather_body,
        mesh=plsc.VectorSubcoreMesh(core_axis_name="c", subcore_axis_name="s"),
        out_type=jax.ShapeDtypeStruct((E_PAD, c), jnp.float32),
        scratch_types=[
            pltpu.VMEM((_GK,), jnp.int32),
            pltpu.VMEM((_GK,), jnp.int32),
            pltpu.VMEM((_GK, c), jnp.float32),
            pltpu.VMEM((_GK, c), jnp.float32),
            pltpu.SemaphoreType.DMA,
            pltpu.SemaphoreType.DMA,
        ],
    )
    return fn(A, Q, srcp, dstp)


def _node_prep_kernel(ivf_ref, xyzp_ref, w1a_ref, w1b_ref, b1_ref, a_ref, q_ref):
    q = jnp.dot(xyzp_ref[...], w1b_ref[...], preferred_element_type=jnp.float32)
    a = jnp.dot(ivf_ref[...], w1a_ref[...], preferred_element_type=jnp.float32)
    a_ref[...] = a + q + b1_ref[...]
    q_ref[...] = q


def _node_prep(ivf, xyzp, w1aT, w1bpT, b1):
    n = ivf.shape[0]
    c = ivf.shape[1]
    grid = (n // N_BLK,)
    return pl.pallas_call(
        _node_prep_kernel,
        grid=grid,
        in_specs=[
            pl.BlockSpec((N_BLK, c), lambda i: (i, 0)),
            pl.BlockSpec((N_BLK, 128), lambda i: (i, 0)),
            pl.BlockSpec((c, c), lambda i: (0, 0)),
            pl.BlockSpec((128, c), lambda i: (0, 0)),
            pl.BlockSpec((1, c), lambda i: (0, 0)),
        ],
        out_specs=[
            pl.BlockSpec((N_BLK, c), lambda i: (i, 0)),
            pl.BlockSpec((N_BLK, c), lambda i: (i, 0)),
        ],
        out_shape=[
            jax.ShapeDtypeStruct((n, c), jnp.float32),
            jax.ShapeDtypeStruct((n, c), jnp.float32),
        ],
    )(ivf, xyzp, w1aT, w1bpT, b1)


_NEG = -3.0e38
_NR = 320
_DCH = 640
_SCH = 250


def _scatter_body(h1_hbm, dst_hbm, out_hbm, dstbuf, rowsbuf, agg_l, ridx, sem):
    wid = lax.axis_index("s") * _NC + lax.axis_index("c")
    lo = wid * _NR

    def init(i, carry):
        agg_l[pl.ds(i * 16, 16)] = jnp.full((16,), _NEG, dtype=jnp.float32)
        return carry

    lax.fori_loop(0, _NR * 256 // 16, init, 0)

    def chunk(j, carry):
        e0 = j * _DCH
        pltpu.sync_copy(dst_hbm.at[pl.ds(e0, _DCH)], dstbuf)

        def sub(s, carry_s):
            def grp(t, cnt_t):
                dvec = dstbuf[pl.ds(s * _GK + t * 16, 16)]
                c2 = cnt_t
                for l in range(16):
                    d = dvec[l]
                    hit = (d >= lo) & (d < lo + _NR)

                    @pl.when(hit)
                    def _(d=d, l=l, c2=c2):
                        pltpu.async_copy(
                            h1_hbm.at[pl.ds(
                                (e0 + s * _GK + t * 16 + l) * 256, 256)],
                            rowsbuf.at[pl.ds(c2 * 256, 256)], sem)
                        ridx[c2] = d - lo

                    c2 = jnp.where(hit, c2 + 1, c2)
                return c2

            cnt = lax.fori_loop(0, _GK // 16, grp, 0)

            def drain(k, carry2):
                pltpu.make_async_copy(h1_hbm.at[pl.ds(0, 256)],
                                      rowsbuf.at[pl.ds(0, 256)], sem).wait()
                return carry2

            lax.fori_loop(0, cnt, drain, 0)

            def acc(k, carry3):
                r = ridx[k]
                for i in range(16):
                    cur = agg_l[pl.ds(r * 256 + i * 16, 16)]
                    v = rowsbuf[pl.ds(k * 256 + i * 16, 16)]
                    agg_l[pl.ds(r * 256 + i * 16, 16)] = jnp.where(
                        v > cur, v, cur)
                return carry3

            lax.fori_loop(0, cnt, acc, 0)
            return carry_s

        lax.fori_loop(0, _DCH // _GK, sub, 0)
        return carry

    lax.fori_loop(0, _SCH, chunk, 0)
    pltpu.sync_copy(agg_l, out_hbm.at[pl.ds(wid * _NR * 256, _NR * 256)])


def _scatter_max(h, dst):
    h1 = h.reshape(E_PAD * 256)
    fn = pl.kernel(
        _scatter_body,
        mesh=plsc.VectorSubcoreMesh(core_axis_name="c", subcore_axis_name="s"),
        out_type=jax.ShapeDtypeStruct((32 * _NR * 256,), jnp.float32),
        scratch_types=[
            pltpu.VMEM((_DCH,), jnp.int32),
            pltpu.VMEM((_GK * 256,), jnp.float32),
            pltpu.VMEM((_NR * 256,), jnp.float32),
            pltpu.SMEM((_GK,), jnp.int32),
            pltpu.SemaphoreType.DMA,
        ],
    )
    raw = fn(h1, dst)
    return raw.reshape(32 * _NR, 256)[:10000]


def _edge_ln_kernel(diff_ref, g_ref, be_ref, h_ref):
    x = jnp.maximum(diff_ref[...], 0.0)
    mu = jnp.mean(x, axis=-1, keepdims=True)
    xc = x - mu
    var = jnp.mean(xc * xc, axis=-1, keepdims=True)
    h_ref[...] = xc * jax.lax.rsqrt(var + 1e-5) * g_ref[...] + be_ref[...]


def _edge_ln(diff, g1, be1):
    e, c = diff.shape
    return pl.pallas_call(
        _edge_ln_kernel,
        grid=(e // E_BLK,),
        in_specs=[
            pl.BlockSpec((E_BLK, c), lambda i: (i, 0)),
            pl.BlockSpec((1, c), lambda i: (0, 0)),
            pl.BlockSpec((1, c), lambda i: (0, 0)),
        ],
        out_specs=pl.BlockSpec((E_BLK, c), lambda i: (i, 0)),
        out_shape=jax.ShapeDtypeStruct((e, c), jnp.float32),
    )(diff, g1, be1)


def _update_kernel(agg_ref, ivf_ref, w2_ref, b2_ref, g_ref, be_ref, out_ref):
    a = agg_ref[...]
    a = jnp.where(a < -1.0e38, 0.0, a)
    u = jnp.dot(a, w2_ref[...], preferred_element_type=jnp.float32)
    u = jnp.maximum(u + b2_ref[...], 0.0)
    mu = jnp.mean(u, axis=-1, keepdims=True)
    uc = u - mu
    var = jnp.mean(uc * uc, axis=-1, keepdims=True)
    out_ref[...] = uc * jax.lax.rsqrt(var + 1e-5) * g_ref[...] + be_ref[...] + ivf_ref[...]


def _update(agg, ivf, w2T, b2, g2, be2):
    n, c = agg.shape
    return pl.pallas_call(
        _update_kernel,
        grid=(n // N_BLK,),
        in_specs=[
            pl.BlockSpec((N_BLK, c), lambda i: (i, 0)),
            pl.BlockSpec((N_BLK, c), lambda i: (i, 0)),
            pl.BlockSpec((c, c), lambda i: (0, 0)),
            pl.BlockSpec((1, c), lambda i: (0, 0)),
            pl.BlockSpec((1, c), lambda i: (0, 0)),
            pl.BlockSpec((1, c), lambda i: (0, 0)),
        ],
        out_specs=pl.BlockSpec((N_BLK, c), lambda i: (i, 0)),
        out_shape=jax.ShapeDtypeStruct((n, c), jnp.float32),
    )(agg, ivf, w2T, b2, g2, be2)


def kernel(xyz, features, edges, W1, b1, g1, be1, W2, b2, g2, be2):
    n = xyz.shape[1]
    c = features.shape[1]
    ivf = features[0].T
    src = edges[0, :, 0]
    dst = edges[0, :, 1]
    xyzp = jnp.pad(xyz[0], ((0, 0), (0, 125)))
    w1aT = W1[:, :c].T
    w1bpT = jnp.pad(W1[:, c:].T, ((0, 125), (0, 0)))

    A, Q = _node_prep(ivf, xyzp, w1aT, w1bpT, b1[None, :])

    e = src.shape[0]
    srcp = jnp.pad(src, (0, E_PAD - e))
    dstp = jnp.pad(dst, (0, E_PAD - e))
    diff = _edge_gather(A, Q, srcp, dstp)
    h = _edge_ln(diff, g1[None, :], be1[None, :])
    agg = _scatter_max(h, dst)

    u = _update(agg, ivf, W2.T, b2[None, :], g2[None, :], be2[None, :])
    return u.T[None]
